# Initial kernel scaffold; baseline (speedup 1.0000x reference)
import jax
import jax.numpy as jnp
from jax.experimental import pallas as pl


def kernel(x, blocks, W):
    raise NotImplementedError("write your pallas kernel here")


# speedup vs baseline: 3.0307x; 3.0307x over previous
import jax
import jax.numpy as jnp
from jax.experimental import pallas as pl


BT = 1024


def _body(b_ref, x_ref, w_ref, o_ref):
    b = b_ref[0, 0, :]
    onehot = (b[:, None] == jax.lax.broadcasted_iota(jnp.int32, (1, 8), 1)).astype(
        jnp.float32
    )
    emb = jnp.dot(onehot, w_ref[:, :], preferred_element_type=jnp.float32)
    o_ref[:, :] = x_ref[:, :] + emb


def kernel(x, blocks, W):
    B, T, D = x.shape
    N = B * T
    x2 = x.reshape(N, D)
    b3 = blocks.reshape(N // BT, 1, BT).astype(jnp.int32)
    grid = (N // BT,)
    out = pl.pallas_call(
        _body,
        grid=grid,
        in_specs=[
            pl.BlockSpec((1, 1, BT), lambda i: (i, 0, 0)),
            pl.BlockSpec((BT, D), lambda i: (i, 0)),
            pl.BlockSpec((8, D), lambda i: (0, 0)),
        ],
        out_specs=pl.BlockSpec((BT, D), lambda i: (i, 0)),
        out_shape=jax.ShapeDtypeStruct((N, D), jnp.float32),
    )(b3, x2, W)
    return out.reshape(B, T, D)
